# baseline (speedup 1.0000x reference)
import functools

import numpy as np
import jax
import jax.numpy as jnp
from jax import lax
from jax.experimental import pallas as pl
from jax.experimental.pallas import tpu as pltpu
from jax.experimental.pallas import tpu_sc as plsc

B, T, H = 16, 512, 256
MAXM = 2048
DPC = 384
VPC, VPK, VPL = 384, 5, 5
NG = 4
GB = B // NG
ZROW = GB * T
GTBL = GB * T + 144

SC_NW = 32
GROWS = GB * MAXM
RPW = GROWS // SC_NW
CHUNK = 128
NCHUNK = RPW // CHUNK


def _pe_np():
    half = H // 2
    emb = np.float32(np.log(10000.0) / (half - 1))
    inv = np.exp(np.arange(half, dtype=np.float32) * -emb).astype(np.float32)
    pos = np.arange(1, T + 1, dtype=np.float32)
    ang = pos[:, None] * inv[None, :]
    return np.concatenate([np.sin(ang), np.cos(ang)], axis=1).astype(np.float32)


_PE = _pe_np()
_TRIU = np.triu(np.ones((T, T), dtype=np.float32))


def _ln(h, g2, b2):
    m = jnp.mean(h, axis=1, keepdims=True)
    v = jnp.mean((h - m) ** 2, axis=1, keepdims=True)
    return (h - m) * lax.rsqrt(v + 1e-5) * g2 + b2


def _conv(xin, w_ref, k):
    padn = (k - 1) // 2
    z = jnp.zeros((padn, xin.shape[1]), jnp.float32)
    xp = jnp.concatenate([z, xin, z], axis=0)
    acc = xp[0:T] @ w_ref[0]
    for s in range(1, k):
        acc = acc + xp[s:s + T] @ w_ref[s]
    return acc


def _dp_body(x_ref, dpw0, dpb0, dpg0, dpt0, dpw1, dpb1, dpg1, dpt1,
             dplw, dplb, logd_ref):
    x = x_ref[0]
    h = _conv(x, dpw0, 3) + dpb0[...]
    h = jnp.maximum(h, 0.0)
    h = _ln(h, dpg0[...], dpt0[...])
    h = _conv(h, dpw1, 3) + dpb1[...]
    h = jnp.maximum(h, 0.0)
    h = _ln(h, dpg1[...], dpt1[...])
    logd_ref[...] = (h @ dplw[...] + dplb[...])[None]


def _vp_body(maxlen_ref, x_ref, dur_ref, pe_ref, triu_ref,
             alpha, vw0, vw1, vw2, vw3, vw4, vb, vg, vt, vlw, vlb, pw, ew,
             var_ref, xv_ref, idx_ref, mel_ref):
    x = x_ref[0]

    hv = x + alpha[...] * pe_ref[...]
    vws = (vw0, vw1, vw2, vw3, vw4)
    vbv, vgv, vtv = vb[...], vg[...], vt[...]
    for i in range(VPL):
        hv = _conv(hv, vws[i], VPK) + vbv[i:i + 1, :]
        hv = jnp.maximum(hv, 0.0)
        hv = _ln(hv, vgv[i:i + 1, :], vtv[i:i + 1, :])
    var2 = hv @ vlw[...] + vlb[...]
    var_ref[...] = var2[None]

    xv = x + var2[:, 0:1] * pw[...] + var2[:, 1:2] * ew[...]
    xv_ref[...] = xv[None]

    b = pl.program_id(0)
    durf = dur_ref[0].astype(jnp.float32)
    cum = durf @ triu_ref[...]
    mel = cum[0:1, T - 1:T]
    maxl = maxlen_ref[0, 0].astype(jnp.float32)
    limit = jnp.minimum(mel, maxl)
    pos = lax.broadcasted_iota(jnp.int32, (MAXM, 1), 0).astype(jnp.float32)
    cnt = jnp.sum((cum <= pos).astype(jnp.float32), axis=1, keepdims=True)
    idxf = jnp.minimum(cnt, float(T - 1))
    basef = (b * T).astype(jnp.float32)
    flat = jnp.where(pos < limit, idxf + basef, float(ZROW))
    idx_ref[...] = flat.astype(jnp.int32)[None]
    mel_ref[...] = jnp.broadcast_to(
        mel.astype(jnp.int32).reshape(1, 1, 1), (1, 8, 1))


def _wspecs(weights):
    return [pl.BlockSpec(w.shape, lambda b, _r=w.ndim: (0,) * _r)
            for w in weights]


def _vp_call(maxlen_arr, x, dur3, pe, triu, weights):
    return pl.pallas_call(
        _vp_body,
        grid=(GB,),
        in_specs=[
            pl.BlockSpec(memory_space=pltpu.SMEM),
            pl.BlockSpec((1, T, H), lambda b: (b, 0, 0)),
            pl.BlockSpec((1, 1, T), lambda b: (b, 0, 0)),
            pl.BlockSpec((T, H), lambda b: (0, 0)),
            pl.BlockSpec((T, T), lambda b: (0, 0)),
        ] + _wspecs(weights),
        out_specs=[
            pl.BlockSpec((1, T, 2), lambda b: (b, 0, 0)),
            pl.BlockSpec((1, T, H), lambda b: (b, 0, 0)),
            pl.BlockSpec((1, MAXM, 1), lambda b: (b, 0, 0)),
            pl.BlockSpec((1, 8, 1), lambda b: (b, 0, 0)),
        ],
        out_shape=[
            jax.ShapeDtypeStruct((GB, T, 2), jnp.float32),
            jax.ShapeDtypeStruct((GB, T, H), jnp.float32),
            jax.ShapeDtypeStruct((GB, MAXM, 1), jnp.int32),
            jax.ShapeDtypeStruct((GB, 8, 1), jnp.int32),
        ],
    )(maxlen_arr, x, dur3, pe, triu, *weights)


def _dp_call(x, weights):
    return pl.pallas_call(
        _dp_body,
        grid=(B,),
        in_specs=[pl.BlockSpec((1, T, H), lambda b: (b, 0, 0))]
        + _wspecs(weights),
        out_specs=pl.BlockSpec((1, T, 1), lambda b: (b, 0, 0)),
        out_shape=jax.ShapeDtypeStruct((B, T, 1), jnp.float32),
    )(x, *weights)


def _sc_gather(table, idx):
    mesh = plsc.VectorSubcoreMesh(core_axis_name="c", subcore_axis_name="s")

    @functools.partial(
        pl.kernel,
        out_type=jax.ShapeDtypeStruct((GROWS, H), jnp.float32),
        mesh=mesh,
        scratch_types=[
            pltpu.VMEM((RPW + 16,), jnp.int32),
            pltpu.VMEM((CHUNK + 8, H), jnp.float32),
            pltpu.VMEM((CHUNK, H), jnp.float32),
            pltpu.VMEM((CHUNK, H), jnp.float32),
            pltpu.SemaphoreType.DMA,
            pltpu.SemaphoreType.DMA,
            pltpu.SemaphoreType.DMA,
        ],
    )
    def k(table_hbm, idx_hbm, out_hbm, idx_v, win_v, st0, st1,
          gsem, ss0, ss1):
        wid = lax.axis_index("s") * 2 + lax.axis_index("c")
        base = wid * RPW
        pltpu.sync_copy(idx_hbm.at[pl.ds(base, RPW)], idx_v.at[pl.ds(0, RPW)])
        stages = (st0, st1)
        ssems = (ss0, ss1)
        ds = [None, None]
        for c in range(NCHUNK):
            bi = c % 2
            stage = stages[bi]
            lo = idx_v[pl.ds(c * CHUNK, 16)][0]
            hi = idx_v[pl.ds(c * CHUNK + CHUNK - 16, 16)][15]
            if ds[bi] is not None:
                ds[bi].wait()

            lo8 = pl.multiple_of(lo - lax.rem(lo, 8), 8)

            @pl.when(hi - lo < CHUNK)
            def _fast():
                pltpu.sync_copy(table_hbm.at[pl.ds(lo8, CHUNK + 8)], win_v)

                def body(g, _):
                    g16 = g * 16
                    locv = idx_v[pl.ds(c * CHUNK + g16, 16)] - lo8
                    for r in range(16):
                        loc = locv[r]
                        for j in range(H // 16):
                            stage[g16 + r, pl.ds(j * 16, 16)] = \
                                win_v[loc, pl.ds(j * 16, 16)]
                    return 0

                lax.fori_loop(0, CHUNK // 16, body, 0)

            @pl.when(hi - lo >= CHUNK)
            def _slow():
                pltpu.async_copy(
                    table_hbm.at[idx_v.at[pl.ds(c * CHUNK, CHUNK)]],
                    stage, gsem).wait()

            ds[bi] = pltpu.async_copy(
                stage, out_hbm.at[pl.ds(base + c * CHUNK, CHUNK)], ssems[bi])
        ds[0].wait()
        ds[1].wait()

    return k(table, idx)


def kernel(x, duration, max_len, params):
    p = params
    maxlen_arr = jnp.asarray(max_len, jnp.int32).reshape(1, 1)
    dur3 = duration.reshape(B, 1, T)
    pe = jnp.asarray(_PE)
    triu = jnp.asarray(_TRIU)
    r2 = lambda a: a.reshape(1, -1)
    vp_weights = [
        r2(p['vp_pos_alpha']),
        p['vp_conv_w'][0], p['vp_conv_w'][1], p['vp_conv_w'][2],
        p['vp_conv_w'][3], p['vp_conv_w'][4],
        jnp.stack(p['vp_conv_b']), jnp.stack(p['vp_ln_g']), jnp.stack(p['vp_ln_b']),
        p['vp_lin_w'], r2(p['vp_lin_b']),
        p['pitch_emb_w'], p['energy_emb_w'],
    ]
    dp_weights = [
        p['dp_conv0_w'], r2(p['dp_conv0_b']), r2(p['dp_ln0_g']), r2(p['dp_ln0_b']),
        p['dp_conv1_w'], r2(p['dp_conv1_b']), r2(p['dp_ln1_g']), r2(p['dp_ln1_b']),
        p['dp_lin_w'], r2(p['dp_lin_b']),
    ]
    vars_, mels, outs = [], [], []
    zpad = jnp.zeros((144, H), jnp.float32)
    for g in range(NG):
        x_g = lax.slice_in_dim(x, g * GB, (g + 1) * GB)
        d_g = lax.slice_in_dim(dur3, g * GB, (g + 1) * GB)
        var_g, xv_g, idx_g, mel_g = _vp_call(maxlen_arr, x_g, d_g, pe, triu,
                                             vp_weights)
        table_g = jnp.concatenate([xv_g.reshape(GB * T, H), zpad], axis=0)
        outs.append(_sc_gather(table_g, idx_g.reshape(GROWS)))
        vars_.append(var_g)
        mels.append(mel_g)
    logd3 = _dp_call(x, dp_weights)
    out = jnp.concatenate(outs, axis=0).reshape(B, MAXM, H)
    var = jnp.concatenate(vars_, axis=0)
    mel3 = jnp.concatenate(mels, axis=0)
    return out, logd3[..., 0], var, mel3[:, 0, 0]


# speedup vs baseline: 1.0774x; 1.0774x over previous
import functools

import numpy as np
import jax
import jax.numpy as jnp
from jax import lax
from jax.experimental import pallas as pl
from jax.experimental.pallas import tpu as pltpu
from jax.experimental.pallas import tpu_sc as plsc

B, T, H = 16, 512, 256
MAXM = 2048
DPC = 384
VPC, VPK, VPL = 384, 5, 5
NG = 2
GB = B // NG
ZROW = GB * T
GTBL = GB * T + 144

SC_NW = 32
GROWS = GB * MAXM
RPW = GROWS // SC_NW
CHUNK = 128
NCHUNK = RPW // CHUNK


def _pe_np():
    half = H // 2
    emb = np.float32(np.log(10000.0) / (half - 1))
    inv = np.exp(np.arange(half, dtype=np.float32) * -emb).astype(np.float32)
    pos = np.arange(1, T + 1, dtype=np.float32)
    ang = pos[:, None] * inv[None, :]
    return np.concatenate([np.sin(ang), np.cos(ang)], axis=1).astype(np.float32)


_PE = _pe_np()
_TRIU = np.triu(np.ones((T, T), dtype=np.float32))


def _ln(h, g2, b2):
    m = jnp.mean(h, axis=1, keepdims=True)
    v = jnp.mean((h - m) ** 2, axis=1, keepdims=True)
    return (h - m) * lax.rsqrt(v + 1e-5) * g2 + b2


def _conv(xin, w_ref, k):
    padn = (k - 1) // 2
    xb = xin.astype(jnp.bfloat16)
    z = jnp.zeros((padn, xin.shape[1]), jnp.bfloat16)
    xp = jnp.concatenate([z, xb, z], axis=0)
    acc = jnp.dot(xp[0:T], w_ref[0], preferred_element_type=jnp.float32)
    for s in range(1, k):
        acc = acc + jnp.dot(xp[s:s + T], w_ref[s],
                            preferred_element_type=jnp.float32)
    return acc


def _dp_body(x_ref, dpw0, dpb0, dpg0, dpt0, dpw1, dpb1, dpg1, dpt1,
             dplw, dplb, logd_ref):
    x = x_ref[0]
    h = _conv(x, dpw0, 3) + dpb0[...]
    h = jnp.maximum(h, 0.0)
    h = _ln(h, dpg0[...], dpt0[...])
    h = _conv(h, dpw1, 3) + dpb1[...]
    h = jnp.maximum(h, 0.0)
    h = _ln(h, dpg1[...], dpt1[...])
    logd_ref[...] = (h @ dplw[...] + dplb[...])[None]


def _vp_body(maxlen_ref, x_ref, dur_ref, pe_ref, triu_ref,
             alpha, vw0, vw1, vw2, vw3, vw4, vb, vg, vt, vlw, vlb, pw, ew,
             var_ref, xv_ref, idx_ref, mel_ref):
    x = x_ref[0]

    hv = x + alpha[...] * pe_ref[...]
    vws = (vw0, vw1, vw2, vw3, vw4)
    vbv, vgv, vtv = vb[...], vg[...], vt[...]
    for i in range(VPL):
        hv = _conv(hv, vws[i], VPK) + vbv[i:i + 1, :]
        hv = jnp.maximum(hv, 0.0)
        hv = _ln(hv, vgv[i:i + 1, :], vtv[i:i + 1, :])
    var2 = hv @ vlw[...] + vlb[...]
    var_ref[...] = var2[None]

    xv = x + var2[:, 0:1] * pw[...] + var2[:, 1:2] * ew[...]
    xv_ref[...] = xv[None]

    b = pl.program_id(0)
    durf = dur_ref[0].astype(jnp.float32)
    cum = durf @ triu_ref[...]
    mel = cum[0:1, T - 1:T]
    maxl = maxlen_ref[0, 0].astype(jnp.float32)
    limit = jnp.minimum(mel, maxl)
    pos = lax.broadcasted_iota(jnp.int32, (MAXM, 1), 0).astype(jnp.float32)
    cnt = jnp.sum((cum <= pos).astype(jnp.float32), axis=1, keepdims=True)
    idxf = jnp.minimum(cnt, float(T - 1))
    basef = (b * T).astype(jnp.float32)
    flat = jnp.where(pos < limit, idxf + basef, float(ZROW))
    idx_ref[...] = flat.astype(jnp.int32)[None]
    mel_ref[...] = jnp.broadcast_to(
        mel.astype(jnp.int32).reshape(1, 1, 1), (1, 8, 1))


def _wspecs(weights):
    return [pl.BlockSpec(w.shape, lambda b, _r=w.ndim: (0,) * _r)
            for w in weights]


def _vp_call(maxlen_arr, x, dur3, pe, triu, weights):
    return pl.pallas_call(
        _vp_body,
        grid=(GB,),
        in_specs=[
            pl.BlockSpec(memory_space=pltpu.SMEM),
            pl.BlockSpec((1, T, H), lambda b: (b, 0, 0)),
            pl.BlockSpec((1, 1, T), lambda b: (b, 0, 0)),
            pl.BlockSpec((T, H), lambda b: (0, 0)),
            pl.BlockSpec((T, T), lambda b: (0, 0)),
        ] + _wspecs(weights),
        out_specs=[
            pl.BlockSpec((1, T, 2), lambda b: (b, 0, 0)),
            pl.BlockSpec((1, T, H), lambda b: (b, 0, 0)),
            pl.BlockSpec((1, MAXM, 1), lambda b: (b, 0, 0)),
            pl.BlockSpec((1, 8, 1), lambda b: (b, 0, 0)),
        ],
        out_shape=[
            jax.ShapeDtypeStruct((GB, T, 2), jnp.float32),
            jax.ShapeDtypeStruct((GB, T, H), jnp.float32),
            jax.ShapeDtypeStruct((GB, MAXM, 1), jnp.int32),
            jax.ShapeDtypeStruct((GB, 8, 1), jnp.int32),
        ],
    )(maxlen_arr, x, dur3, pe, triu, *weights)


def _dp_call(x, weights):
    return pl.pallas_call(
        _dp_body,
        grid=(B,),
        in_specs=[pl.BlockSpec((1, T, H), lambda b: (b, 0, 0))]
        + _wspecs(weights),
        out_specs=pl.BlockSpec((1, T, 1), lambda b: (b, 0, 0)),
        out_shape=jax.ShapeDtypeStruct((B, T, 1), jnp.float32),
    )(x, *weights)


def _sc_gather(table, idx):
    mesh = plsc.VectorSubcoreMesh(core_axis_name="c", subcore_axis_name="s")

    @functools.partial(
        pl.kernel,
        out_type=jax.ShapeDtypeStruct((GROWS, H), jnp.float32),
        mesh=mesh,
        scratch_types=[
            pltpu.VMEM((RPW + 16,), jnp.int32),
            pltpu.VMEM((CHUNK + 8, H), jnp.float32),
            pltpu.VMEM((CHUNK, H), jnp.float32),
            pltpu.VMEM((CHUNK, H), jnp.float32),
            pltpu.SemaphoreType.DMA,
            pltpu.SemaphoreType.DMA,
            pltpu.SemaphoreType.DMA,
        ],
    )
    def k(table_hbm, idx_hbm, out_hbm, idx_v, win_v, st0, st1,
          gsem, ss0, ss1):
        wid = lax.axis_index("s") * 2 + lax.axis_index("c")
        base = wid * RPW
        pltpu.sync_copy(idx_hbm.at[pl.ds(base, RPW)], idx_v.at[pl.ds(0, RPW)])
        stages = (st0, st1)
        ssems = (ss0, ss1)
        ds = [None, None]
        for c in range(NCHUNK):
            bi = c % 2
            stage = stages[bi]
            lo = idx_v[pl.ds(c * CHUNK, 16)][0]
            hi = idx_v[pl.ds(c * CHUNK + CHUNK - 16, 16)][15]
            if ds[bi] is not None:
                ds[bi].wait()

            lo8 = pl.multiple_of(lo - lax.rem(lo, 8), 8)

            @pl.when(hi - lo < CHUNK)
            def _fast():
                pltpu.sync_copy(table_hbm.at[pl.ds(lo8, CHUNK + 8)], win_v)

                def body(g, _):
                    g16 = g * 16
                    locv = idx_v[pl.ds(c * CHUNK + g16, 16)] - lo8
                    for r in range(16):
                        loc = locv[r]
                        for j in range(H // 16):
                            stage[g16 + r, pl.ds(j * 16, 16)] = \
                                win_v[loc, pl.ds(j * 16, 16)]
                    return 0

                lax.fori_loop(0, CHUNK // 16, body, 0)

            @pl.when(hi - lo >= CHUNK)
            def _slow():
                pltpu.async_copy(
                    table_hbm.at[idx_v.at[pl.ds(c * CHUNK, CHUNK)]],
                    stage, gsem).wait()

            ds[bi] = pltpu.async_copy(
                stage, out_hbm.at[pl.ds(base + c * CHUNK, CHUNK)], ssems[bi])
        ds[0].wait()
        ds[1].wait()

    return k(table, idx)


def kernel(x, duration, max_len, params):
    p = params
    maxlen_arr = jnp.asarray(max_len, jnp.int32).reshape(1, 1)
    dur3 = duration.reshape(B, 1, T)
    pe = jnp.asarray(_PE)
    triu = jnp.asarray(_TRIU)
    r2 = lambda a: a.reshape(1, -1)
    bf = lambda a: a.astype(jnp.bfloat16)
    vp_weights = [
        r2(p['vp_pos_alpha']),
        bf(p['vp_conv_w'][0]), bf(p['vp_conv_w'][1]), bf(p['vp_conv_w'][2]),
        bf(p['vp_conv_w'][3]), bf(p['vp_conv_w'][4]),
        jnp.stack(p['vp_conv_b']), jnp.stack(p['vp_ln_g']), jnp.stack(p['vp_ln_b']),
        p['vp_lin_w'], r2(p['vp_lin_b']),
        p['pitch_emb_w'], p['energy_emb_w'],
    ]
    dp_weights = [
        bf(p['dp_conv0_w']), r2(p['dp_conv0_b']), r2(p['dp_ln0_g']), r2(p['dp_ln0_b']),
        bf(p['dp_conv1_w']), r2(p['dp_conv1_b']), r2(p['dp_ln1_g']), r2(p['dp_ln1_b']),
        p['dp_lin_w'], r2(p['dp_lin_b']),
    ]
    vars_, mels, outs = [], [], []
    zpad = jnp.zeros((144, H), jnp.float32)
    for g in range(NG):
        x_g = lax.slice_in_dim(x, g * GB, (g + 1) * GB)
        d_g = lax.slice_in_dim(dur3, g * GB, (g + 1) * GB)
        var_g, xv_g, idx_g, mel_g = _vp_call(maxlen_arr, x_g, d_g, pe, triu,
                                             vp_weights)
        table_g = jnp.concatenate([xv_g.reshape(GB * T, H), zpad], axis=0)
        outs.append(_sc_gather(table_g, idx_g.reshape(GROWS)))
        vars_.append(var_g)
        mels.append(mel_g)
    logd3 = _dp_call(x, dp_weights)
    out = jnp.concatenate(outs, axis=0).reshape(B, MAXM, H)
    var = jnp.concatenate(vars_, axis=0)
    mel3 = jnp.concatenate(mels, axis=0)
    return out, logd3[..., 0], var, mel3[:, 0, 0]


# speedup vs baseline: 1.0925x; 1.0140x over previous
import functools

import numpy as np
import jax
import jax.numpy as jnp
from jax import lax
from jax.experimental import pallas as pl
from jax.experimental.pallas import tpu as pltpu
from jax.experimental.pallas import tpu_sc as plsc

B, T, H = 16, 512, 256
MAXM = 2048
DPC = 384
VPC, VPK, VPL = 384, 5, 5
NG = 2
GB = B // NG
ZROW = GB * T
GTBL = GB * T + 144

SC_NW = 32
GROWS = GB * MAXM
RPW = GROWS // SC_NW
CHUNK = 128
NCHUNK = RPW // CHUNK


def _pe_np():
    half = H // 2
    emb = np.float32(np.log(10000.0) / (half - 1))
    inv = np.exp(np.arange(half, dtype=np.float32) * -emb).astype(np.float32)
    pos = np.arange(1, T + 1, dtype=np.float32)
    ang = pos[:, None] * inv[None, :]
    return np.concatenate([np.sin(ang), np.cos(ang)], axis=1).astype(np.float32)


_PE = _pe_np()
_TRIU = np.triu(np.ones((T, T), dtype=np.float32))


def _ln(h, g2, b2):
    m = jnp.mean(h, axis=1, keepdims=True)
    v = jnp.mean((h - m) ** 2, axis=1, keepdims=True)
    return (h - m) * lax.rsqrt(v + 1e-5) * g2 + b2


def _conv(xin, w_ref, k):
    padn = (k - 1) // 2
    z = jnp.zeros((padn, xin.shape[1]), jnp.float32)
    xp = jnp.concatenate([z, xin, z], axis=0)
    acc = xp[0:T] @ w_ref[0]
    for s in range(1, k):
        acc = acc + xp[s:s + T] @ w_ref[s]
    return acc


def _dp_body(x_ref, dpw0, dpb0, dpg0, dpt0, dpw1, dpb1, dpg1, dpt1,
             dplw, dplb, logd_ref):
    x = x_ref[0]
    h = _conv(x, dpw0, 3) + dpb0[...]
    h = jnp.maximum(h, 0.0)
    h = _ln(h, dpg0[...], dpt0[...])
    h = _conv(h, dpw1, 3) + dpb1[...]
    h = jnp.maximum(h, 0.0)
    h = _ln(h, dpg1[...], dpt1[...])
    logd_ref[...] = (h @ dplw[...] + dplb[...])[None]


def _vp_body(maxlen_ref, x_ref, dur_ref, pe_ref, triu_ref,
             alpha, vw0, vw1, vw2, vw3, vw4, vb, vg, vt, vlw, vlb, pw, ew,
             var_ref, xv_ref, idx_ref, mel_ref):
    x = x_ref[0]

    hv = x + alpha[...] * pe_ref[...]
    vws = (vw0, vw1, vw2, vw3, vw4)
    vbv, vgv, vtv = vb[...], vg[...], vt[...]
    for i in range(VPL):
        hv = _conv(hv, vws[i], VPK) + vbv[i:i + 1, :]
        hv = jnp.maximum(hv, 0.0)
        hv = _ln(hv, vgv[i:i + 1, :], vtv[i:i + 1, :])
    var2 = hv @ vlw[...] + vlb[...]
    var_ref[...] = var2[None]

    xv = x + var2[:, 0:1] * pw[...] + var2[:, 1:2] * ew[...]
    xv_ref[...] = xv[None]

    b = pl.program_id(0)
    durf = dur_ref[0].astype(jnp.float32)
    cum = durf @ triu_ref[...]
    mel = cum[0:1, T - 1:T]
    maxl = maxlen_ref[0, 0].astype(jnp.float32)
    limit = jnp.minimum(mel, maxl)
    pos = lax.broadcasted_iota(jnp.int32, (MAXM, 1), 0).astype(jnp.float32)
    cnt = jnp.sum((cum <= pos).astype(jnp.float32), axis=1, keepdims=True)
    idxf = jnp.minimum(cnt, float(T - 1))
    basef = (b * T).astype(jnp.float32)
    flat = jnp.where(pos < limit, idxf + basef, float(ZROW))
    idx_ref[...] = flat.astype(jnp.int32)[None]
    mel_ref[...] = jnp.broadcast_to(
        mel.astype(jnp.int32).reshape(1, 1, 1), (1, 8, 1))


def _wspecs(weights):
    return [pl.BlockSpec(w.shape, lambda b, _r=w.ndim: (0,) * _r)
            for w in weights]


def _vp_call(maxlen_arr, x, dur3, pe, triu, weights):
    return pl.pallas_call(
        _vp_body,
        grid=(GB,),
        in_specs=[
            pl.BlockSpec(memory_space=pltpu.SMEM),
            pl.BlockSpec((1, T, H), lambda b: (b, 0, 0)),
            pl.BlockSpec((1, 1, T), lambda b: (b, 0, 0)),
            pl.BlockSpec((T, H), lambda b: (0, 0)),
            pl.BlockSpec((T, T), lambda b: (0, 0)),
        ] + _wspecs(weights),
        out_specs=[
            pl.BlockSpec((1, T, 2), lambda b: (b, 0, 0)),
            pl.BlockSpec((1, T, H), lambda b: (b, 0, 0)),
            pl.BlockSpec((1, MAXM, 1), lambda b: (b, 0, 0)),
            pl.BlockSpec((1, 8, 1), lambda b: (b, 0, 0)),
        ],
        out_shape=[
            jax.ShapeDtypeStruct((GB, T, 2), jnp.float32),
            jax.ShapeDtypeStruct((GB, T, H), jnp.float32),
            jax.ShapeDtypeStruct((GB, MAXM, 1), jnp.int32),
            jax.ShapeDtypeStruct((GB, 8, 1), jnp.int32),
        ],
    )(maxlen_arr, x, dur3, pe, triu, *weights)


def _dp_call(x, weights):
    return pl.pallas_call(
        _dp_body,
        grid=(B,),
        in_specs=[pl.BlockSpec((1, T, H), lambda b: (b, 0, 0))]
        + _wspecs(weights),
        out_specs=pl.BlockSpec((1, T, 1), lambda b: (b, 0, 0)),
        out_shape=jax.ShapeDtypeStruct((B, T, 1), jnp.float32),
    )(x, *weights)


def _sc_gather(table, idx):
    mesh = plsc.VectorSubcoreMesh(core_axis_name="c", subcore_axis_name="s")

    @functools.partial(
        pl.kernel,
        out_type=jax.ShapeDtypeStruct((GROWS, H), jnp.float32),
        mesh=mesh,
        scratch_types=[
            pltpu.VMEM((RPW + 16,), jnp.int32),
            pltpu.VMEM((CHUNK + 8, H), jnp.float32),
            pltpu.VMEM((CHUNK, H), jnp.float32),
            pltpu.VMEM((CHUNK, H), jnp.float32),
            pltpu.SemaphoreType.DMA,
            pltpu.SemaphoreType.DMA,
            pltpu.SemaphoreType.DMA,
        ],
    )
    def k(table_hbm, idx_hbm, out_hbm, idx_v, win_v, st0, st1,
          gsem, ss0, ss1):
        wid = lax.axis_index("s") * 2 + lax.axis_index("c")
        base = wid * RPW
        pltpu.sync_copy(idx_hbm.at[pl.ds(base, RPW)], idx_v.at[pl.ds(0, RPW)])
        stages = (st0, st1)
        ssems = (ss0, ss1)
        ds = [None, None]
        for c in range(NCHUNK):
            bi = c % 2
            stage = stages[bi]
            lo = idx_v[pl.ds(c * CHUNK, 16)][0]
            hi = idx_v[pl.ds(c * CHUNK + CHUNK - 16, 16)][15]
            if ds[bi] is not None:
                ds[bi].wait()

            lo8 = pl.multiple_of(lo - lax.rem(lo, 8), 8)

            @pl.when(hi - lo < CHUNK)
            def _fast():
                pltpu.sync_copy(table_hbm.at[pl.ds(lo8, CHUNK + 8)], win_v)

                def body(g, _):
                    g16 = g * 16
                    locv = idx_v[pl.ds(c * CHUNK + g16, 16)] - lo8
                    for r in range(16):
                        loc = locv[r]
                        for j in range(H // 16):
                            stage[g16 + r, pl.ds(j * 16, 16)] = \
                                win_v[loc, pl.ds(j * 16, 16)]
                    return 0

                lax.fori_loop(0, CHUNK // 16, body, 0)

            @pl.when(hi - lo >= CHUNK)
            def _slow():
                pltpu.async_copy(
                    table_hbm.at[idx_v.at[pl.ds(c * CHUNK, CHUNK)]],
                    stage, gsem).wait()

            ds[bi] = pltpu.async_copy(
                stage, out_hbm.at[pl.ds(base + c * CHUNK, CHUNK)], ssems[bi])
        ds[0].wait()
        ds[1].wait()

    return k(table, idx)


def kernel(x, duration, max_len, params):
    p = params
    maxlen_arr = jnp.asarray(max_len, jnp.int32).reshape(1, 1)
    dur3 = duration.reshape(B, 1, T)
    pe = jnp.asarray(_PE)
    triu = jnp.asarray(_TRIU)
    r2 = lambda a: a.reshape(1, -1)
    vp_weights = [
        r2(p['vp_pos_alpha']),
        p['vp_conv_w'][0], p['vp_conv_w'][1], p['vp_conv_w'][2],
        p['vp_conv_w'][3], p['vp_conv_w'][4],
        jnp.stack(p['vp_conv_b']), jnp.stack(p['vp_ln_g']), jnp.stack(p['vp_ln_b']),
        p['vp_lin_w'], r2(p['vp_lin_b']),
        p['pitch_emb_w'], p['energy_emb_w'],
    ]
    dp_weights = [
        p['dp_conv0_w'], r2(p['dp_conv0_b']), r2(p['dp_ln0_g']), r2(p['dp_ln0_b']),
        p['dp_conv1_w'], r2(p['dp_conv1_b']), r2(p['dp_ln1_g']), r2(p['dp_ln1_b']),
        p['dp_lin_w'], r2(p['dp_lin_b']),
    ]
    vars_, mels, outs = [], [], []
    zpad = jnp.zeros((144, H), jnp.float32)
    for g in range(NG):
        x_g = lax.slice_in_dim(x, g * GB, (g + 1) * GB)
        d_g = lax.slice_in_dim(dur3, g * GB, (g + 1) * GB)
        var_g, xv_g, idx_g, mel_g = _vp_call(maxlen_arr, x_g, d_g, pe, triu,
                                             vp_weights)
        table_g = jnp.concatenate([xv_g.reshape(GB * T, H), zpad], axis=0)
        outs.append(_sc_gather(table_g, idx_g.reshape(GROWS)))
        vars_.append(var_g)
        mels.append(mel_g)
    logd3 = _dp_call(x, dp_weights)
    out = jnp.concatenate(outs, axis=0).reshape(B, MAXM, H)
    var = jnp.concatenate(vars_, axis=0)
    mel3 = jnp.concatenate(mels, axis=0)
    return out, logd3[..., 0], var, mel3[:, 0, 0]


# speedup vs baseline: 1.0972x; 1.0043x over previous
import functools

import numpy as np
import jax
import jax.numpy as jnp
from jax import lax
from jax.experimental import pallas as pl
from jax.experimental.pallas import tpu as pltpu
from jax.experimental.pallas import tpu_sc as plsc

B, T, H = 16, 512, 256
MAXM = 2048
DPC = 384
VPC, VPK, VPL = 384, 5, 5
NG = 2
GB = B // NG
ZROW = GB * T
GTBL = GB * T + 144

SC_NW = 32
GROWS = GB * MAXM
RPW = GROWS // SC_NW
CHUNK = 128
NCHUNK = RPW // CHUNK


def _pe_np():
    half = H // 2
    emb = np.float32(np.log(10000.0) / (half - 1))
    inv = np.exp(np.arange(half, dtype=np.float32) * -emb).astype(np.float32)
    pos = np.arange(1, T + 1, dtype=np.float32)
    ang = pos[:, None] * inv[None, :]
    return np.concatenate([np.sin(ang), np.cos(ang)], axis=1).astype(np.float32)


_PE = _pe_np()
_TRIU = np.triu(np.ones((T, T), dtype=np.float32))


def _ln(h, g2, b2):
    m = jnp.mean(h, axis=1, keepdims=True)
    v = jnp.mean((h - m) ** 2, axis=1, keepdims=True)
    return (h - m) * lax.rsqrt(v + 1e-5) * g2 + b2


def _conv(xin, w_ref, k):
    padn = (k - 1) // 2
    z = jnp.zeros((padn, xin.shape[1]), jnp.float32)
    xp = jnp.concatenate([z, xin, z], axis=0)
    acc = xp[0:T] @ w_ref[0]
    for s in range(1, k):
        acc = acc + xp[s:s + T] @ w_ref[s]
    return acc


def _dp_body(x_ref, dpw0, dpb0, dpg0, dpt0, dpw1, dpb1, dpg1, dpt1,
             dplw, dplb, logd_ref):
    x = x_ref[0]
    h = _conv(x, dpw0, 3) + dpb0[...]
    h = jnp.maximum(h, 0.0)
    h = _ln(h, dpg0[...], dpt0[...])
    h = _conv(h, dpw1, 3) + dpb1[...]
    h = jnp.maximum(h, 0.0)
    h = _ln(h, dpg1[...], dpt1[...])
    logd_ref[...] = (h @ dplw[...] + dplb[...])[None]


def _vp_body(maxlen_ref, x_ref, dur_ref, pe_ref, triu_ref,
             alpha, vw0, vw1, vw2, vw3, vw4, vb, vg, vt, vlw, vlb, pw, ew,
             var_ref, xv_ref, idx_ref, mel_ref):
    x = x_ref[0]

    hv = x + alpha[...] * pe_ref[...]
    vws = (vw0, vw1, vw2, vw3, vw4)
    vbv, vgv, vtv = vb[...], vg[...], vt[...]
    for i in range(VPL):
        hv = _conv(hv, vws[i], VPK) + vbv[i:i + 1, :]
        hv = jnp.maximum(hv, 0.0)
        hv = _ln(hv, vgv[i:i + 1, :], vtv[i:i + 1, :])
    var2 = hv @ vlw[...] + vlb[...]
    var_ref[...] = var2[None]

    xv = x + var2[:, 0:1] * pw[...] + var2[:, 1:2] * ew[...]
    xv_ref[...] = xv[None]

    b = pl.program_id(0)
    durf = dur_ref[0].astype(jnp.float32)
    cum = durf @ triu_ref[...]
    mel = cum[0:1, T - 1:T]
    maxl = maxlen_ref[0, 0].astype(jnp.float32)
    limit = jnp.minimum(mel, maxl)
    pos = lax.broadcasted_iota(jnp.int32, (MAXM, 1), 0).astype(jnp.float32)
    cnt = jnp.sum((cum <= pos).astype(jnp.float32), axis=1, keepdims=True)
    idxf = jnp.minimum(cnt, float(T - 1))
    basef = (b * T).astype(jnp.float32)
    flat = jnp.where(pos < limit, idxf + basef, float(ZROW))
    idx_ref[...] = flat.astype(jnp.int32)[None]
    mel_ref[...] = jnp.broadcast_to(
        mel.astype(jnp.int32).reshape(1, 1, 1), (1, 8, 1))


def _wspecs(weights):
    return [pl.BlockSpec(w.shape, lambda b, _r=w.ndim: (0,) * _r)
            for w in weights]


def _vp_call(maxlen_arr, x, dur3, pe, triu, weights):
    return pl.pallas_call(
        _vp_body,
        grid=(GB,),
        in_specs=[
            pl.BlockSpec(memory_space=pltpu.SMEM),
            pl.BlockSpec((1, T, H), lambda b: (b, 0, 0)),
            pl.BlockSpec((1, 1, T), lambda b: (b, 0, 0)),
            pl.BlockSpec((T, H), lambda b: (0, 0)),
            pl.BlockSpec((T, T), lambda b: (0, 0)),
        ] + _wspecs(weights),
        out_specs=[
            pl.BlockSpec((1, T, 2), lambda b: (b, 0, 0)),
            pl.BlockSpec((1, T, H), lambda b: (b, 0, 0)),
            pl.BlockSpec((1, MAXM, 1), lambda b: (b, 0, 0)),
            pl.BlockSpec((1, 8, 1), lambda b: (b, 0, 0)),
        ],
        out_shape=[
            jax.ShapeDtypeStruct((GB, T, 2), jnp.float32),
            jax.ShapeDtypeStruct((GB, T, H), jnp.float32),
            jax.ShapeDtypeStruct((GB, MAXM, 1), jnp.int32),
            jax.ShapeDtypeStruct((GB, 8, 1), jnp.int32),
        ],
    )(maxlen_arr, x, dur3, pe, triu, *weights)


def _dp_call(x, weights):
    return pl.pallas_call(
        _dp_body,
        grid=(B,),
        in_specs=[pl.BlockSpec((1, T, H), lambda b: (b, 0, 0))]
        + _wspecs(weights),
        out_specs=pl.BlockSpec((1, T, 1), lambda b: (b, 0, 0)),
        out_shape=jax.ShapeDtypeStruct((B, T, 1), jnp.float32),
    )(x, *weights)


def _sc_gather(table, idx):
    mesh = plsc.VectorSubcoreMesh(core_axis_name="c", subcore_axis_name="s")

    @functools.partial(
        pl.kernel,
        out_type=jax.ShapeDtypeStruct((GROWS, H), jnp.float32),
        mesh=mesh,
        scratch_types=[
            pltpu.VMEM((RPW + 16,), jnp.int32),
            pltpu.VMEM((CHUNK + 8, H), jnp.float32),
            pltpu.VMEM((CHUNK + 8, H), jnp.float32),
            pltpu.VMEM((CHUNK, H), jnp.float32),
            pltpu.SemaphoreType.DMA,
            pltpu.SemaphoreType.DMA,
            pltpu.SemaphoreType.DMA,
            pltpu.SemaphoreType.DMA,
        ],
    )
    def k(table_hbm, idx_hbm, out_hbm, idx_v, win0, win1, stage,
          gsem, ws0, ws1, ssem):
        wid = lax.axis_index("s") * 2 + lax.axis_index("c")
        base = wid * RPW
        pltpu.sync_copy(idx_hbm.at[pl.ds(base, RPW)], idx_v.at[pl.ds(0, RPW)])
        wins = (win0, win1)
        wsems = (ws0, ws1)

        def lo8_of(c):
            lo = idx_v[pl.ds(c * CHUNK, 16)][0]
            return pl.multiple_of(lo - lax.rem(lo, 8), 8)

        def wload(c):
            return pltpu.async_copy(
                table_hbm.at[pl.ds(lo8_of(c), CHUNK + 8)],
                wins[c % 2], wsems[c % 2])

        dw = [wload(0), wload(1)]
        ds = None
        for c in range(NCHUNK):
            wi = c % 2
            win_v = wins[wi]
            lo = idx_v[pl.ds(c * CHUNK, 16)][0]
            hi = idx_v[pl.ds(c * CHUNK + CHUNK - 16, 16)][15]
            lo8 = pl.multiple_of(lo - lax.rem(lo, 8), 8)
            dw[wi].wait()
            if ds is not None:
                ds.wait()

            @pl.when(hi - lo < CHUNK)
            def _fast():
                def body(g, _):
                    g16 = g * 16
                    locv = idx_v[pl.ds(c * CHUNK + g16, 16)] - lo8
                    for r in range(16):
                        loc = locv[r]
                        for j in range(H // 16):
                            stage[g16 + r, pl.ds(j * 16, 16)] = \
                                win_v[loc, pl.ds(j * 16, 16)]
                    return 0

                lax.fori_loop(0, CHUNK // 16, body, 0)

            @pl.when(hi - lo >= CHUNK)
            def _slow():
                pltpu.async_copy(
                    table_hbm.at[idx_v.at[pl.ds(c * CHUNK, CHUNK)]],
                    stage, gsem).wait()

            if c + 2 < NCHUNK:
                dw[wi] = wload(c + 2)
            ds = pltpu.async_copy(
                stage, out_hbm.at[pl.ds(base + c * CHUNK, CHUNK)], ssem)
        ds.wait()

    return k(table, idx)


def kernel(x, duration, max_len, params):
    p = params
    maxlen_arr = jnp.asarray(max_len, jnp.int32).reshape(1, 1)
    dur3 = duration.reshape(B, 1, T)
    pe = jnp.asarray(_PE)
    triu = jnp.asarray(_TRIU)
    r2 = lambda a: a.reshape(1, -1)
    vp_weights = [
        r2(p['vp_pos_alpha']),
        p['vp_conv_w'][0], p['vp_conv_w'][1], p['vp_conv_w'][2],
        p['vp_conv_w'][3], p['vp_conv_w'][4],
        jnp.stack(p['vp_conv_b']), jnp.stack(p['vp_ln_g']), jnp.stack(p['vp_ln_b']),
        p['vp_lin_w'], r2(p['vp_lin_b']),
        p['pitch_emb_w'], p['energy_emb_w'],
    ]
    dp_weights = [
        p['dp_conv0_w'], r2(p['dp_conv0_b']), r2(p['dp_ln0_g']), r2(p['dp_ln0_b']),
        p['dp_conv1_w'], r2(p['dp_conv1_b']), r2(p['dp_ln1_g']), r2(p['dp_ln1_b']),
        p['dp_lin_w'], r2(p['dp_lin_b']),
    ]
    vars_, mels, outs = [], [], []
    zpad = jnp.zeros((144, H), jnp.float32)
    for g in range(NG):
        x_g = lax.slice_in_dim(x, g * GB, (g + 1) * GB)
        d_g = lax.slice_in_dim(dur3, g * GB, (g + 1) * GB)
        var_g, xv_g, idx_g, mel_g = _vp_call(maxlen_arr, x_g, d_g, pe, triu,
                                             vp_weights)
        table_g = jnp.concatenate([xv_g.reshape(GB * T, H), zpad], axis=0)
        outs.append(_sc_gather(table_g, idx_g.reshape(GROWS)))
        vars_.append(var_g)
        mels.append(mel_g)
    logd3 = _dp_call(x, dp_weights)
    out = jnp.concatenate(outs, axis=0).reshape(B, MAXM, H)
    var = jnp.concatenate(vars_, axis=0)
    mel3 = jnp.concatenate(mels, axis=0)
    return out, logd3[..., 0], var, mel3[:, 0, 0]


# speedup vs baseline: 1.1099x; 1.0116x over previous
import functools

import numpy as np
import jax
import jax.numpy as jnp
from jax import lax
from jax.experimental import pallas as pl
from jax.experimental.pallas import tpu as pltpu
from jax.experimental.pallas import tpu_sc as plsc

B, T, H = 16, 512, 256
MAXM = 2048
DPC = 384
VPC, VPK, VPL = 384, 5, 5
NG = 2
GB = B // NG
SROW = T + 8
GTBL = GB * SROW + 144

SC_NW = 32
GROWS = GB * MAXM
RPW = GROWS // SC_NW
CHUNK = 128
NCHUNK = RPW // CHUNK


def _pe_np():
    half = H // 2
    emb = np.float32(np.log(10000.0) / (half - 1))
    inv = np.exp(np.arange(half, dtype=np.float32) * -emb).astype(np.float32)
    pos = np.arange(1, T + 1, dtype=np.float32)
    ang = pos[:, None] * inv[None, :]
    return np.concatenate([np.sin(ang), np.cos(ang)], axis=1).astype(np.float32)


_PE = _pe_np()
_TRIU = np.triu(np.ones((T, T), dtype=np.float32))


def _ln(h, g2, b2):
    m = jnp.mean(h, axis=1, keepdims=True)
    v = jnp.mean((h - m) ** 2, axis=1, keepdims=True)
    return (h - m) * lax.rsqrt(v + 1e-5) * g2 + b2


def _conv(xin, w_ref, k):
    padn = (k - 1) // 2
    z = jnp.zeros((padn, xin.shape[1]), jnp.float32)
    xp = jnp.concatenate([z, xin, z], axis=0)
    acc = xp[0:T] @ w_ref[0]
    for s in range(1, k):
        acc = acc + xp[s:s + T] @ w_ref[s]
    return acc


def _dp_body(x_ref, dpw0, dpb0, dpg0, dpt0, dpw1, dpb1, dpg1, dpt1,
             dplw, dplb, logd_ref):
    x = x_ref[0]
    h = _conv(x, dpw0, 3) + dpb0[...]
    h = jnp.maximum(h, 0.0)
    h = _ln(h, dpg0[...], dpt0[...])
    h = _conv(h, dpw1, 3) + dpb1[...]
    h = jnp.maximum(h, 0.0)
    h = _ln(h, dpg1[...], dpt1[...])
    logd_ref[...] = (h @ dplw[...] + dplb[...])[None]


def _vp_body(maxlen_ref, x_ref, dur_ref, pe_ref, triu_ref,
             alpha, vw0, vw1, vw2, vw3, vw4, vb, vg, vt, vlw, vlb, pw, ew,
             var_ref, xv_ref, idx_ref, mel_ref):
    x = x_ref[0]

    hv = x + alpha[...] * pe_ref[...]
    vws = (vw0, vw1, vw2, vw3, vw4)
    vbv, vgv, vtv = vb[...], vg[...], vt[...]
    for i in range(VPL):
        hv = _conv(hv, vws[i], VPK) + vbv[i:i + 1, :]
        hv = jnp.maximum(hv, 0.0)
        hv = _ln(hv, vgv[i:i + 1, :], vtv[i:i + 1, :])
    var2 = hv @ vlw[...] + vlb[...]
    var_ref[...] = var2[None]

    xv = x + var2[:, 0:1] * pw[...] + var2[:, 1:2] * ew[...]
    xv_ref[...] = xv[None]

    b = pl.program_id(0)
    durf = dur_ref[0].astype(jnp.float32)
    cum = durf @ triu_ref[...]
    mel = cum[0:1, T - 1:T]
    maxl = maxlen_ref[0, 0].astype(jnp.float32)
    limit = jnp.minimum(mel, maxl)
    pos = lax.broadcasted_iota(jnp.int32, (MAXM, 1), 0).astype(jnp.float32)
    cnt = jnp.sum((cum <= pos).astype(jnp.float32), axis=1, keepdims=True)
    idxf = jnp.minimum(cnt, float(T - 1))
    basef = (b * SROW).astype(jnp.float32)
    flat = jnp.where(pos < limit, idxf + basef, basef + float(T))
    idx_ref[...] = flat.astype(jnp.int32)[None]
    mel_ref[...] = jnp.broadcast_to(
        mel.astype(jnp.int32).reshape(1, 1, 1), (1, 8, 1))


def _wspecs(weights):
    return [pl.BlockSpec(w.shape, lambda b, _r=w.ndim: (0,) * _r)
            for w in weights]


def _vp_call(maxlen_arr, x, dur3, pe, triu, weights):
    return pl.pallas_call(
        _vp_body,
        grid=(GB,),
        in_specs=[
            pl.BlockSpec(memory_space=pltpu.SMEM),
            pl.BlockSpec((1, T, H), lambda b: (b, 0, 0)),
            pl.BlockSpec((1, 1, T), lambda b: (b, 0, 0)),
            pl.BlockSpec((T, H), lambda b: (0, 0)),
            pl.BlockSpec((T, T), lambda b: (0, 0)),
        ] + _wspecs(weights),
        out_specs=[
            pl.BlockSpec((1, T, 2), lambda b: (b, 0, 0)),
            pl.BlockSpec((1, T, H), lambda b: (b, 0, 0)),
            pl.BlockSpec((1, MAXM, 1), lambda b: (b, 0, 0)),
            pl.BlockSpec((1, 8, 1), lambda b: (b, 0, 0)),
        ],
        out_shape=[
            jax.ShapeDtypeStruct((GB, T, 2), jnp.float32),
            jax.ShapeDtypeStruct((GB, T, H), jnp.float32),
            jax.ShapeDtypeStruct((GB, MAXM, 1), jnp.int32),
            jax.ShapeDtypeStruct((GB, 8, 1), jnp.int32),
        ],
    )(maxlen_arr, x, dur3, pe, triu, *weights)


def _dp_call(x, weights):
    return pl.pallas_call(
        _dp_body,
        grid=(B,),
        in_specs=[pl.BlockSpec((1, T, H), lambda b: (b, 0, 0))]
        + _wspecs(weights),
        out_specs=pl.BlockSpec((1, T, 1), lambda b: (b, 0, 0)),
        out_shape=jax.ShapeDtypeStruct((B, T, 1), jnp.float32),
    )(x, *weights)


def _sc_gather(table, idx):
    mesh = plsc.VectorSubcoreMesh(core_axis_name="c", subcore_axis_name="s")

    @functools.partial(
        pl.kernel,
        out_type=jax.ShapeDtypeStruct((GROWS, H), jnp.float32),
        mesh=mesh,
        scratch_types=[
            pltpu.VMEM((RPW + 16,), jnp.int32),
            pltpu.VMEM((CHUNK + 8, H), jnp.float32),
            pltpu.VMEM((CHUNK + 8, H), jnp.float32),
            pltpu.VMEM((CHUNK, H), jnp.float32),
            pltpu.SemaphoreType.DMA,
            pltpu.SemaphoreType.DMA,
            pltpu.SemaphoreType.DMA,
            pltpu.SemaphoreType.DMA,
        ],
    )
    def k(table_hbm, idx_hbm, out_hbm, idx_v, win0, win1, stage,
          gsem, ws0, ws1, ssem):
        wid = lax.axis_index("s") * 2 + lax.axis_index("c")
        base = wid * RPW
        pltpu.sync_copy(idx_hbm.at[pl.ds(base, RPW)], idx_v.at[pl.ds(0, RPW)])
        wins = (win0, win1)
        wsems = (ws0, ws1)

        def lo8_of(c):
            lo = idx_v[pl.ds(c * CHUNK, 16)][0]
            return pl.multiple_of(lo - lax.rem(lo, 8), 8)

        def wload(c):
            return pltpu.async_copy(
                table_hbm.at[pl.ds(lo8_of(c), CHUNK + 8)],
                wins[c % 2], wsems[c % 2])

        dw = [wload(0), wload(1)]
        ds = None
        for c in range(NCHUNK):
            wi = c % 2
            win_v = wins[wi]
            lo = idx_v[pl.ds(c * CHUNK, 16)][0]
            hi = idx_v[pl.ds(c * CHUNK + CHUNK - 16, 16)][15]
            lo8 = pl.multiple_of(lo - lax.rem(lo, 8), 8)
            dw[wi].wait()
            if ds is not None:
                ds.wait()

            @pl.when(hi - lo < CHUNK)
            def _fast():
                def body(g, _):
                    g16 = g * 16
                    locv = idx_v[pl.ds(c * CHUNK + g16, 16)] - lo8
                    for r in range(16):
                        loc = locv[r]
                        for j in range(H // 16):
                            stage[g16 + r, pl.ds(j * 16, 16)] = \
                                win_v[loc, pl.ds(j * 16, 16)]
                    return 0

                lax.fori_loop(0, CHUNK // 16, body, 0)

            @pl.when(hi - lo >= CHUNK)
            def _slow():
                pltpu.async_copy(
                    table_hbm.at[idx_v.at[pl.ds(c * CHUNK, CHUNK)]],
                    stage, gsem).wait()

            if c + 2 < NCHUNK:
                dw[wi] = wload(c + 2)
            ds = pltpu.async_copy(
                stage, out_hbm.at[pl.ds(base + c * CHUNK, CHUNK)], ssem)
        ds.wait()

    return k(table, idx)


def kernel(x, duration, max_len, params):
    p = params
    maxlen_arr = jnp.asarray(max_len, jnp.int32).reshape(1, 1)
    dur3 = duration.reshape(B, 1, T)
    pe = jnp.asarray(_PE)
    triu = jnp.asarray(_TRIU)
    r2 = lambda a: a.reshape(1, -1)
    vp_weights = [
        r2(p['vp_pos_alpha']),
        p['vp_conv_w'][0], p['vp_conv_w'][1], p['vp_conv_w'][2],
        p['vp_conv_w'][3], p['vp_conv_w'][4],
        jnp.stack(p['vp_conv_b']), jnp.stack(p['vp_ln_g']), jnp.stack(p['vp_ln_b']),
        p['vp_lin_w'], r2(p['vp_lin_b']),
        p['pitch_emb_w'], p['energy_emb_w'],
    ]
    dp_weights = [
        p['dp_conv0_w'], r2(p['dp_conv0_b']), r2(p['dp_ln0_g']), r2(p['dp_ln0_b']),
        p['dp_conv1_w'], r2(p['dp_conv1_b']), r2(p['dp_ln1_g']), r2(p['dp_ln1_b']),
        p['dp_lin_w'], r2(p['dp_lin_b']),
    ]
    vars_, mels, outs = [], [], []
    zpad = jnp.zeros((144, H), jnp.float32)
    for g in range(NG):
        x_g = lax.slice_in_dim(x, g * GB, (g + 1) * GB)
        d_g = lax.slice_in_dim(dur3, g * GB, (g + 1) * GB)
        var_g, xv_g, idx_g, mel_g = _vp_call(maxlen_arr, x_g, d_g, pe, triu,
                                             vp_weights)
        xv_p = jnp.pad(xv_g, ((0, 0), (0, 8), (0, 0)))
        table_g = jnp.concatenate([xv_p.reshape(GB * SROW, H), zpad], axis=0)
        outs.append(_sc_gather(table_g, idx_g.reshape(GROWS)))
        vars_.append(var_g)
        mels.append(mel_g)
    logd3 = _dp_call(x, dp_weights)
    out = jnp.concatenate(outs, axis=0).reshape(B, MAXM, H)
    var = jnp.concatenate(vars_, axis=0)
    mel3 = jnp.concatenate(mels, axis=0)
    return out, logd3[..., 0], var, mel3[:, 0, 0]
